# baseline (speedup 1.0000x reference)
import jax
import jax.numpy as jnp
from jax.experimental import pallas as pl

_N = 65536
_D = 64
_K = 1024
_BR = 2048


def _vq_block(x_ref, w_ref, out_ref):
    x = x_ref[...]
    w = w_ref[...]
    xw = jax.lax.dot_general(
        x, w, (((1,), (1,)), ((), ())), preferred_element_type=jnp.float32)
    d = (jnp.sum(x * x, axis=1, keepdims=True)
         + jnp.sum(w * w, axis=1)[None, :]
         - 2.0 * xw)
    idx = jnp.argmin(d, axis=1).astype(jnp.int32)
    out_ref[...] = idx.reshape(out_ref.shape)


def kernel(x, W):
    grid = _N // _BR
    out = pl.pallas_call(
        _vq_block,
        grid=(grid,),
        in_specs=[
            pl.BlockSpec((_BR, _D), lambda i: (i, 0)),
            pl.BlockSpec((_K, _D), lambda i: (0, 0)),
        ],
        out_specs=pl.BlockSpec((_BR // 128, 128), lambda i: (i, 0)),
        out_shape=jax.ShapeDtypeStruct((_N // 128, 128), jnp.int32),
    )(x, W)
    return out.reshape(_N)


# speedup vs baseline: 1.4051x; 1.4051x over previous
import jax
import jax.numpy as jnp
from jax.experimental import pallas as pl
from jax.experimental.pallas import tpu as pltpu

_N = 65536
_D = 64
_K = 1024
_BR = 2048
_C = _K // 128


def _vq_block(x_ref, w_ref, out_ref, hw_ref):
    @pl.when(pl.program_id(0) == 0)
    def _init():
        w0 = w_ref[...]
        hw_ref[...] = (0.5 * jnp.sum(w0 * w0, axis=1)).reshape(_C, 128)

    x = x_ref[...]
    xw = jax.lax.dot_general(
        x, w_ref[...], (((1,), (1,)), ((), ())),
        preferred_element_type=jnp.float32)
    hw = hw_ref[...]

    val = hw[0:1, :] - xw[:, 0:128]
    bidx = jnp.zeros((_BR, 128), jnp.int32)
    for b in range(1, _C):
        sb = hw[b:b + 1, :] - xw[:, b * 128:(b + 1) * 128]
        m = sb < val
        val = jnp.where(m, sb, val)
        bidx = jnp.where(m, jnp.int32(b), bidx)

    rowmin = jnp.min(val, axis=1, keepdims=True)
    lane = jax.lax.broadcasted_iota(jnp.int32, (_BR, 128), 1)
    j = bidx * 128 + lane
    cand = jnp.where(val == rowmin, j, jnp.int32(1 << 30))
    idx = jnp.min(cand, axis=1).astype(jnp.int32)
    out_ref[...] = idx.reshape(out_ref.shape)


def kernel(x, W):
    grid = _N // _BR
    out = pl.pallas_call(
        _vq_block,
        grid=(grid,),
        in_specs=[
            pl.BlockSpec((_BR, _D), lambda i: (i, 0)),
            pl.BlockSpec((_K, _D), lambda i: (0, 0)),
        ],
        out_specs=pl.BlockSpec((_BR // 128, 128), lambda i: (i, 0)),
        out_shape=jax.ShapeDtypeStruct((_N // 128, 128), jnp.int32),
        scratch_shapes=[pltpu.VMEM((_C, 128), jnp.float32)],
    )(x, W)
    return out.reshape(_N)


# speedup vs baseline: 1.6970x; 1.2077x over previous
import jax
import jax.numpy as jnp
from jax.experimental import pallas as pl
from jax.experimental.pallas import tpu as pltpu

_N = 65536
_D = 64
_K = 1024
_BR = 2048
_C = _K // 128


def _vq_block(x_ref, w_ref, out_ref, wa_ref):
    @pl.when(pl.program_id(0) == 0)
    def _init():
        w0 = w_ref[...]
        hw = 0.5 * jnp.sum(w0 * w0, axis=1, keepdims=True)
        wa_ref[...] = jnp.concatenate([-w0, hw], axis=1)

    x = x_ref[...]
    xa = jnp.concatenate([x, jnp.ones((_BR, 1), jnp.float32)], axis=1)
    s = jax.lax.dot_general(
        xa, wa_ref[...], (((1,), (1,)), ((), ())),
        preferred_element_type=jnp.float32)

    val = s[:, 0:128]
    bidx = jnp.zeros((_BR, 128), jnp.float32)
    for b in range(1, _C):
        sb = s[:, b * 128:(b + 1) * 128]
        m = sb < val
        val = jnp.where(m, sb, val)
        bidx = jnp.where(m, jnp.float32(b), bidx)

    rowmin = jnp.min(val, axis=1, keepdims=True)
    lane = jax.lax.broadcasted_iota(
        jnp.int32, (_BR, 128), 1).astype(jnp.float32)
    j = bidx * 128.0 + lane
    cand = jnp.where(val == rowmin, j, jnp.float32(2.0 ** 30))
    idx = jnp.min(cand, axis=1).astype(jnp.int32)
    out_ref[...] = idx.reshape(out_ref.shape)


def kernel(x, W):
    grid = _N // _BR
    out = pl.pallas_call(
        _vq_block,
        grid=(grid,),
        in_specs=[
            pl.BlockSpec((_BR, _D), lambda i: (i, 0)),
            pl.BlockSpec((_K, _D), lambda i: (0, 0)),
        ],
        out_specs=pl.BlockSpec((_BR // 128, 128), lambda i: (i, 0)),
        out_shape=jax.ShapeDtypeStruct((_N // 128, 128), jnp.int32),
        scratch_shapes=[pltpu.VMEM((_K, _D + 1), jnp.float32)],
    )(x, W)
    return out.reshape(_N)
